# baseline (speedup 1.0000x reference)
import functools

import jax
import jax.numpy as jnp
from jax import lax
from jax.experimental import pallas as pl
from jax.experimental.pallas import tpu as pltpu
from jax.experimental.pallas import tpu_sc as plsc

_K = 4
_NTILES = 32
_SUB = 112


def _gate_kernel(x_ref, gw_ref, gb_ref, idx_ref, val_ref, *, k, e, hw):
    n_id = pl.program_id(0)
    xb = x_ref[0]
    gw = gw_ref[...]
    gb = gb_ref[...]
    logits = jnp.dot(gw, xb, preferred_element_type=jnp.float32) + gb
    m = jnp.max(logits, axis=0, keepdims=True)
    p = jnp.exp(logits - m)
    rw = p / jnp.sum(p, axis=0, keepdims=True)

    rows = lax.broadcasted_iota(jnp.int32, (e, hw), 0)
    s_iota = lax.broadcasted_iota(jnp.int32, (1, hw), 1)
    cur = rw
    fids = []
    vals = []
    for _ in range(k):
        mx = jnp.max(cur, axis=0, keepdims=True)
        sel = cur == mx
        first = jnp.min(jnp.where(sel, rows, e), axis=0, keepdims=True)
        sel = rows == first
        fids.append((n_id * e + first) * hw + s_iota)
        vals.append(mx)
        cur = jnp.where(sel, -1.0, cur)
    idx_ref[0] = jnp.concatenate(fids, axis=0)
    val_ref[0] = jnp.concatenate(vals, axis=0)


def _dispatch_kernel(table_hbm, idx_hbm, val_hbm, out_hbm,
                     idx_v, val_v, rows_v, acc_v, sem, *, hw, rows_per_tile):
    k = _K
    sub = _SUB
    nsub = rows_per_tile // sub
    wid = lax.axis_index("s") * 2 + lax.axis_index("c")
    qpt = hw // rows_per_tile
    n_id = wid // qpt
    q_id = wid % qpt

    for i in range(k):
        src = (n_id * k + i) * hw + q_id * rows_per_tile
        pltpu.sync_copy(idx_hbm.at[pl.ds(src, rows_per_tile)],
                        idx_v.at[pl.ds(i * rows_per_tile, rows_per_tile)])
        pltpu.sync_copy(val_hbm.at[pl.ds(src, rows_per_tile)],
                        val_v.at[pl.ds(i * rows_per_tile, rows_per_tile)])

    out_base = n_id * hw + q_id * rows_per_tile

    def sub_body(c, carry):
        copies = [
            pltpu.async_copy(
                table_hbm.at[idx_v.at[pl.ds(i * rows_per_tile + c * sub, sub)]],
                rows_v.at[i], sem)
            for i in range(k)
        ]
        for cp in copies:
            cp.wait()
        for g in range(sub // 16):
            vgs = [val_v[pl.ds(i * rows_per_tile + c * sub + g * 16, 16)]
                   for i in range(k)]
            for j in range(16):
                r = g * 16 + j
                jj = jnp.full((16,), j, dtype=jnp.int32)
                acc = None
                for i in range(k):
                    vs = jnp.take(vgs[i], jj)
                    for dj in range(4):
                        term = rows_v[i, r, pl.ds(dj * 16, 16)] * vs
                        if acc is None:
                            acc = [term]
                        elif len(acc) <= dj:
                            acc.append(term)
                        else:
                            acc[dj] = acc[dj] + term
                for dj in range(4):
                    acc_v[r, pl.ds(dj * 16, 16)] = acc[dj]
        pltpu.sync_copy(acc_v,
                        out_hbm.at[pl.ds(out_base + c * sub, sub)])
        return carry

    lax.fori_loop(0, nsub, sub_body, 0)


def kernel(x, experts, gate_w, gate_b):
    n, c, h, w = x.shape
    _, e, _, _, d = experts.shape
    k = _K
    hw = h * w
    rows_per_tile = (n * hw) // _NTILES

    xr = x.reshape(n, c, hw)
    gb = gate_b.reshape(e, 1)

    idx, vals = pl.pallas_call(
        functools.partial(_gate_kernel, k=k, e=e, hw=hw),
        grid=(n,),
        in_specs=[
            pl.BlockSpec((1, c, hw), lambda i: (i, 0, 0)),
            pl.BlockSpec((e, c), lambda i: (0, 0)),
            pl.BlockSpec((e, 1), lambda i: (0, 0)),
        ],
        out_specs=[
            pl.BlockSpec((1, k, hw), lambda i: (i, 0, 0)),
            pl.BlockSpec((1, k, hw), lambda i: (i, 0, 0)),
        ],
        out_shape=[
            jax.ShapeDtypeStruct((n, k, hw), jnp.int32),
            jax.ShapeDtypeStruct((n, k, hw), jnp.float32),
        ],
    )(xr, gate_w, gb)

    table = experts.reshape(n * e * hw, d)
    idx_flat = idx.reshape(n * k * hw)
    val_flat = vals.reshape(n * k * hw)

    mesh = plsc.VectorSubcoreMesh(core_axis_name="c", subcore_axis_name="s")
    disp = functools.partial(
        pl.kernel,
        out_type=jax.ShapeDtypeStruct((n * hw, d), jnp.float32),
        mesh=mesh,
        scratch_types=[
            pltpu.VMEM((k * rows_per_tile,), jnp.int32),
            pltpu.VMEM((k * rows_per_tile,), jnp.float32),
            pltpu.VMEM((k, _SUB, d), jnp.float32),
            pltpu.VMEM((_SUB, d), jnp.float32),
            pltpu.SemaphoreType.DMA,
        ],
        compiler_params=pltpu.CompilerParams(use_tc_tiling_on_sc=False),
    )(functools.partial(_dispatch_kernel, hw=hw, rows_per_tile=rows_per_tile))

    out = disp(table, idx_flat, val_flat)
    return out.reshape(n, h, w, d)


# speedup vs baseline: 2.1606x; 2.1606x over previous
import functools

import jax
import jax.numpy as jnp
from jax.experimental import pallas as pl
from jax.experimental.pallas import tpu as pltpu


def _moe_kernel(x_ref, ex_ref, gw_ref, gb_ref, out_ref, wt_ref, *, k, sb):
    s_idx = pl.program_id(1)

    @pl.when(s_idx == 0)
    def _gate():
        xb = x_ref[0]
        gw = gw_ref[...]
        gb = gb_ref[...]
        e = gw.shape[0]
        hw = xb.shape[1]
        logits = jnp.dot(gw, xb, preferred_element_type=jnp.float32) + gb
        m = jnp.max(logits, axis=0, keepdims=True)
        p = jnp.exp(logits - m)
        rw = p / jnp.sum(p, axis=0, keepdims=True)

        rows = jax.lax.broadcasted_iota(jnp.int32, (e, hw), 0)
        cur = rw
        wsel = jnp.zeros_like(rw)
        for _ in range(k):
            mx = jnp.max(cur, axis=0, keepdims=True)
            sel = cur == mx
            first = jnp.min(jnp.where(sel, rows, e), axis=0, keepdims=True)
            sel = rows == first
            wsel = wsel + jnp.where(sel, rw, 0.0)
            cur = jnp.where(sel, -1.0, cur)
        wt_ref[...] = wsel.T

    e = gw_ref.shape[0]
    wt = wt_ref[pl.ds(s_idx * sb, sb), :]
    terms = [wt[:, j:j + 1] * ex_ref[0, j] for j in range(e)]
    while len(terms) > 1:
        terms = [terms[i] + terms[i + 1] for i in range(0, len(terms), 2)]
    out_ref[0] = terms[0]


def kernel(x, experts, gate_w, gate_b):
    n, c, h, w = x.shape
    _, e, _, _, d = experts.shape
    k = 4
    hw = h * w
    sb = 784
    nsb = hw // sb

    xr = x.reshape(n, c, hw)
    er = experts.reshape(n, e, hw, d)
    gb = gate_b.reshape(e, 1)

    out = pl.pallas_call(
        functools.partial(_moe_kernel, k=k, sb=sb),
        grid=(n, nsb),
        in_specs=[
            pl.BlockSpec((1, c, hw), lambda i, s: (i, 0, 0)),
            pl.BlockSpec((1, e, sb, d), lambda i, s: (i, 0, s, 0)),
            pl.BlockSpec((e, c), lambda i, s: (0, 0)),
            pl.BlockSpec((e, 1), lambda i, s: (0, 0)),
        ],
        out_specs=pl.BlockSpec((1, sb, d), lambda i, s: (i, s, 0)),
        out_shape=jax.ShapeDtypeStruct((n, hw, d), jnp.float32),
        scratch_shapes=[pltpu.VMEM((hw, e), jnp.float32)],
    )(xr, er, gate_w, gb)
    return out.reshape(n, h, w, d)


# speedup vs baseline: 2.5369x; 1.1741x over previous
import functools

import jax
import jax.numpy as jnp
from jax.experimental import pallas as pl
from jax.experimental.pallas import tpu as pltpu


def _moe_kernel(x_ref, ex_ref, gw_ref, gb_ref, out_ref, wt_ref, *, k, sb):
    s_idx = pl.program_id(1)

    @pl.when(s_idx == 0)
    def _gate():
        xb = x_ref[0]
        gw = gw_ref[...]
        gb = gb_ref[...]
        e = gw.shape[0]
        hw = xb.shape[1]
        logits = jnp.dot(gw, xb, preferred_element_type=jnp.float32) + gb
        m = jnp.max(logits, axis=0, keepdims=True)
        p = jnp.exp(logits - m)
        rw = p / jnp.sum(p, axis=0, keepdims=True)

        rows = jax.lax.broadcasted_iota(jnp.int32, (e, hw), 0)
        cur = rw
        wsel = jnp.zeros_like(rw)
        for _ in range(k):
            mx = jnp.max(cur, axis=0, keepdims=True)
            sel = cur == mx
            first = jnp.min(jnp.where(sel, rows, e), axis=0, keepdims=True)
            sel = rows == first
            wsel = wsel + jnp.where(sel, rw, 0.0)
            cur = jnp.where(sel, -1.0, cur)
        wt_ref[...] = wsel.T

    e = gw_ref.shape[0]
    d = ex_ref.shape[3]
    wt = wt_ref[pl.ds(s_idx * sb, sb), :]
    re = jax.lax.broadcasted_iota(jnp.int32, (e, 128 * e), 0)
    ce = jax.lax.broadcasted_iota(jnp.int32, (e, 128 * e), 1)
    bmat = ((ce // 128 == re) & (ce % 128 < d)).astype(jnp.float32)
    wtb = jnp.dot(wt, bmat, preferred_element_type=jnp.float32)
    terms = [wtb[:, 128 * j:128 * j + d] * ex_ref[0, j] for j in range(e)]
    while len(terms) > 1:
        terms = [terms[i] + terms[i + 1] for i in range(0, len(terms), 2)]
    out_ref[0] = terms[0]


def kernel(x, experts, gate_w, gate_b):
    n, c, h, w = x.shape
    _, e, _, _, d = experts.shape
    k = 4
    hw = h * w
    sb = 784
    nsb = hw // sb

    xr = x.reshape(n, c, hw)
    er = experts.reshape(n, e, hw, d)
    gb = gate_b.reshape(e, 1)

    out = pl.pallas_call(
        functools.partial(_moe_kernel, k=k, sb=sb),
        grid=(n, nsb),
        in_specs=[
            pl.BlockSpec((1, c, hw), lambda i, s: (i, 0, 0)),
            pl.BlockSpec((1, e, sb, d), lambda i, s: (i, 0, s, 0)),
            pl.BlockSpec((e, c), lambda i, s: (0, 0)),
            pl.BlockSpec((e, 1), lambda i, s: (0, 0)),
        ],
        out_specs=pl.BlockSpec((1, sb, d), lambda i, s: (i, s, 0)),
        out_shape=jax.ShapeDtypeStruct((n, hw, d), jnp.float32),
        scratch_shapes=[pltpu.VMEM((hw, e), jnp.float32)],
    )(xr, er, gate_w, gb)
    return out.reshape(n, h, w, d)
